# baseline (speedup 1.0000x reference)
import functools

import jax
import jax.numpy as jnp
from jax.experimental import pallas as pl
from jax.experimental.pallas import tpu as pltpu

H = 256
W = 256
N = 8192
F = 2
B = 4
NC = 1024
N_CHUNKS = N // NC

_INV_SQRT2 = 0.7071067811865476


def _erf(x):
    ax = jnp.abs(x)
    t = 1.0 / (1.0 + 0.3275911 * ax)
    poly = t * (0.254829592 + t * (-0.284496736 + t * (1.421413741
               + t * (-1.453152027 + t * 1.061405429))))
    y = 1.0 - poly * jnp.exp(-ax * ax)
    return jnp.sign(x) * y


def _inv4(t):
    A2323 = t[2][2] * t[3][3] - t[2][3] * t[3][2]
    A1323 = t[2][1] * t[3][3] - t[2][3] * t[3][1]
    A1223 = t[2][1] * t[3][2] - t[2][2] * t[3][1]
    A0323 = t[2][0] * t[3][3] - t[2][3] * t[3][0]
    A0223 = t[2][0] * t[3][2] - t[2][2] * t[3][0]
    A0123 = t[2][0] * t[3][1] - t[2][1] * t[3][0]
    A2313 = t[1][2] * t[3][3] - t[1][3] * t[3][2]
    A1313 = t[1][1] * t[3][3] - t[1][3] * t[3][1]
    A1213 = t[1][1] * t[3][2] - t[1][2] * t[3][1]
    A2312 = t[1][2] * t[2][3] - t[1][3] * t[2][2]
    A1312 = t[1][1] * t[2][3] - t[1][3] * t[2][1]
    A1212 = t[1][1] * t[2][2] - t[1][2] * t[2][1]
    A0313 = t[1][0] * t[3][3] - t[1][3] * t[3][0]
    A0213 = t[1][0] * t[3][2] - t[1][2] * t[3][0]
    A0312 = t[1][0] * t[2][3] - t[1][3] * t[2][0]
    A0212 = t[1][0] * t[2][2] - t[1][2] * t[2][0]
    A0113 = t[1][0] * t[3][1] - t[1][1] * t[3][0]
    A0112 = t[1][0] * t[2][1] - t[1][1] * t[2][0]

    det = (t[0][0] * (t[1][1] * A2323 - t[1][2] * A1323 + t[1][3] * A1223)
           - t[0][1] * (t[1][0] * A2323 - t[1][2] * A0323 + t[1][3] * A0223)
           + t[0][2] * (t[1][0] * A1323 - t[1][1] * A0323 + t[1][3] * A0123)
           - t[0][3] * (t[1][0] * A1223 - t[1][1] * A0223 + t[1][2] * A0123))
    invdet = 1.0 / det
    m = [[None] * 4 for _ in range(4)]
    m[0][0] = (t[1][1] * A2323 - t[1][2] * A1323 + t[1][3] * A1223) * invdet
    m[0][1] = -(t[0][1] * A2323 - t[0][2] * A1323 + t[0][3] * A1223) * invdet
    m[0][2] = (t[0][1] * A2313 - t[0][2] * A1313 + t[0][3] * A1213) * invdet
    m[0][3] = -(t[0][1] * A2312 - t[0][2] * A1312 + t[0][3] * A1212) * invdet
    m[1][0] = -(t[1][0] * A2323 - t[1][2] * A0323 + t[1][3] * A0223) * invdet
    m[1][1] = (t[0][0] * A2323 - t[0][2] * A0323 + t[0][3] * A0223) * invdet
    m[1][2] = -(t[0][0] * A2313 - t[0][2] * A0313 + t[0][3] * A0213) * invdet
    m[1][3] = (t[0][0] * A2312 - t[0][2] * A0312 + t[0][3] * A0212) * invdet
    m[2][0] = (t[1][0] * A1323 - t[1][1] * A0323 + t[1][3] * A0123) * invdet
    m[2][1] = -(t[0][0] * A1323 - t[0][1] * A0323 + t[0][3] * A0123) * invdet
    m[2][2] = (t[0][0] * A1313 - t[0][1] * A0313 + t[0][3] * A0113) * invdet
    m[2][3] = -(t[0][0] * A1312 - t[0][1] * A0312 + t[0][3] * A0112) * invdet
    m[3][0] = -(t[1][0] * A1223 - t[1][1] * A0223 + t[1][2] * A0123) * invdet
    m[3][1] = (t[0][0] * A1223 - t[0][1] * A0223 + t[0][2] * A0123) * invdet
    m[3][2] = -(t[0][0] * A1213 - t[0][1] * A0213 + t[0][2] * A0113) * invdet
    m[3][3] = (t[0][0] * A1212 - t[0][1] * A0212 + t[0][2] * A0112) * invdet
    return m


def _dense_body(t_ref, crow_ref, ccol_ref, s_ref, st_ref, wt_ref, o_ref):
    b = pl.program_id(0)
    k = pl.program_id(1)
    t = [[t_ref[b, i, j] for j in range(4)] for i in range(4)]
    m = _inv4(t)

    xr = crow_ref[0:1, :]
    yr = crow_ref[1:2, :]
    zr = crow_ref[2:3, :]
    cpw_r = m[3][0] * xr + m[3][1] * yr + m[3][2] * zr + m[3][3]
    cy_r = (m[0][0] * xr + m[0][1] * yr + m[0][2] * zr + m[0][3]) / cpw_r

    xc = ccol_ref[:, 0:1]
    yc = ccol_ref[:, 1:2]
    zc = ccol_ref[:, 2:3]
    cpw_c = m[3][0] * xc + m[3][1] * yc + m[3][2] * zc + m[3][3]
    cx_c = (m[1][0] * xc + m[1][1] * yc + m[1][2] * zc + m[1][3]) / cpw_c

    ys = jax.lax.broadcasted_iota(jnp.int32, (H, 1), 0).astype(jnp.float32)
    xs = jax.lax.broadcasted_iota(jnp.int32, (1, W), 1).astype(jnp.float32)

    acc = jnp.zeros((H, W), jnp.float32)
    for f in range(F):
        s_row = st_ref[f:f + 1, :]
        w_row = wt_ref[f:f + 1, :]
        s_col = s_ref[:, f:f + 1]
        kr = _INV_SQRT2 / s_row
        kc = _INV_SQRT2 / s_col
        iyT = 0.5 * (_erf((ys + 1.0 - cy_r) * kr) - _erf((ys - cy_r) * kr))
        iyT = iyT * w_row
        ix = 0.5 * (_erf((xs + 1.0 - cx_c) * kc) - _erf((xs - cx_c) * kc))
        acc = acc + jax.lax.dot_general(
            iyT, ix, (((1,), (0,)), ((), ())),
            preferred_element_type=jnp.float32)

    @pl.when(k == 0)
    def _():
        o_ref[...] = jnp.zeros_like(o_ref)

    o_ref[...] += acc[None]


@jax.jit
def _dense(transform_matrix, centers, scales, weights):
    centers_t = centers.T
    scales_t = scales.T
    weights_t = weights.T
    grid = (B, N_CHUNKS)
    out = pl.pallas_call(
        _dense_body,
        grid=grid,
        in_specs=[
            pl.BlockSpec(memory_space=pltpu.SMEM),
            pl.BlockSpec((3, NC), lambda b, k: (0, k)),
            pl.BlockSpec((NC, 3), lambda b, k: (k, 0)),
            pl.BlockSpec((NC, F), lambda b, k: (k, 0)),
            pl.BlockSpec((F, NC), lambda b, k: (0, k)),
            pl.BlockSpec((F, NC), lambda b, k: (0, k)),
        ],
        out_specs=pl.BlockSpec((1, H, W), lambda b, k: (b, 0, 0)),
        out_shape=jax.ShapeDtypeStruct((B, H, W), jnp.float32),
    )(transform_matrix, centers_t, centers, scales, scales_t, weights_t)
    return out


def kernel(transform_matrix, centers, scales, weights):
    return _dense(transform_matrix, centers, scales, weights)
